# baseline (speedup 1.0000x reference)
import jax
import jax.numpy as jnp
from jax import lax
from jax.experimental import pallas as pl
from jax.experimental.pallas import tpu as pltpu
from jax.experimental.pallas import tpu_sc as plsc

_EMB_DIM = 128
_N_FIELDS = 26
_VOCAB = 1000
_BATCH = 16384
_LN_EPS = 1e-5

_NC = 2
_NS = 16
_NW = _NC * _NS
_CB = 8
_ROWS_PER_W = _BATCH // _NW
_CHUNKS = _ROWS_PER_W // _CB
_IDX_PER_CHUNK = _N_FIELDS * _CB
_IDX_PER_W = _N_FIELDS * _ROWS_PER_W
_GATHER_SPLIT = 2
_IDX_PER_DMA = _IDX_PER_CHUNK // _GATHER_SPLIT
_NLANE = 16
_NVEC = _EMB_DIM // _NLANE


def _sc_body(t_hbm, idx_hbm, o_hbm, idx_all, rows0, rows1, out0, out1,
             s_t, red_v, sem0, sem1, osem0, osem1):
    wid = lax.axis_index("s") * _NC + lax.axis_index("c")
    first = wid * _CHUNKS
    bufs = ((rows0, sem0), (rows1, sem1))
    obufs = ((out0, osem0), (out1, osem1))

    pltpu.sync_copy(idx_hbm.at[pl.ds(wid * _IDX_PER_W, _IDX_PER_W)],
                    idx_all)

    def fire(local_chunk, p):
        rows_v, sem = bufs[p]
        for g in range(_GATHER_SPLIT):
            off = local_chunk * _IDX_PER_CHUNK + g * _IDX_PER_DMA
            pltpu.async_copy(
                t_hbm.at[idx_all.at[pl.ds(off, _IDX_PER_DMA)]],
                rows_v.at[pl.ds(g * _IDX_PER_DMA, _IDX_PER_DMA)],
                sem)

    def drain(local_chunk, p):
        rows_v, sem = bufs[p]
        for g in range(_GATHER_SPLIT):
            off = local_chunk * _IDX_PER_CHUNK + g * _IDX_PER_DMA
            pltpu.make_async_copy(
                t_hbm.at[idx_all.at[pl.ds(off, _IDX_PER_DMA)]],
                rows_v.at[pl.ds(g * _IDX_PER_DMA, _IDX_PER_DMA)],
                sem).wait()

    fire(0, 0)

    half = jnp.full((_NLANE,), 0.5, jnp.float32)
    three_half = jnp.full((_NLANE,), 1.5, jnp.float32)
    magic = jnp.full((_NLANE,), 0x5F3759DF, jnp.int32)
    one_i = jnp.full((_NLANE,), 1, jnp.int32)
    lanes = lax.iota(jnp.int32, _NLANE)
    lanes_cb = lanes * _CB
    st_idx = [[lanes_cb + (_NLANE * _CB * d + b) for b in range(_CB)]
              for d in range(_NVEC)]
    mean_idx = [jnp.full((_NLANE,), 2 * _NLANE + b, jnp.int32)
                for b in range(_CB)]
    y_idx = [jnp.full((_NLANE,), 3 * _NLANE + b, jnp.int32)
             for b in range(_CB)]
    xor_half = lanes ^ _CB
    xor_half_sq = xor_half + _NLANE

    @pl.loop(0, _CHUNKS, step=2)
    def chunk_loop(i):
        for p in range(2):
            rows_v = bufs[p][0]
            out_v, osem = obufs[p]
            cur = i + p
            nxt = cur + 1

            @pl.when(nxt < _CHUNKS)
            def _():
                fire(nxt, 1 - p)

            drain(cur, p)
            for b in range(_CB):
                base_row = b * _N_FIELDS

                def fbody(f, accs):
                    return tuple(
                        accs[d] + rows_v[base_row + f,
                                         pl.ds(_NLANE * d, _NLANE)]
                        for d in range(_NVEC))
                accs = lax.fori_loop(
                    0, _N_FIELDS, fbody,
                    tuple(jnp.zeros((_NLANE,), jnp.float32)
                          for _ in range(_NVEC)), unroll=4)
                for d in range(_NVEC):
                    plsc.store_scatter(s_t, [st_idx[d][b]], accs[d])
            def sbody(j, ms):
                v = s_t[pl.ds(j * _NLANE, _NLANE)]
                return ms[0] + v, ms[1] + v * v
            msum, msq = lax.fori_loop(
                0, _EMB_DIM * _CB // _NLANE, sbody,
                (jnp.zeros((_NLANE,), jnp.float32),
                 jnp.zeros((_NLANE,), jnp.float32)), unroll=8)
            red_v[pl.ds(0, _NLANE)] = msum
            red_v[pl.ds(_NLANE, _NLANE)] = msq
            msum = msum + plsc.load_gather(red_v, [xor_half])
            msq = msq + plsc.load_gather(red_v, [xor_half_sq])
            mean = msum * (1.0 / _EMB_DIM)
            var = msq * (1.0 / _EMB_DIM) - mean * mean + _LN_EPS
            y = plsc.bitcast(
                magic - lax.shift_right_logical(
                    plsc.bitcast(var, jnp.int32), one_i), jnp.float32)
            hx = half * var
            for _ in range(3):
                y = y * (three_half - hx * y * y)
            red_v[pl.ds(2 * _NLANE, _NLANE)] = mean
            red_v[pl.ds(3 * _NLANE, _NLANE)] = y
            @pl.when(cur >= 2)
            def _():
                pltpu.make_async_copy(
                    out_v, o_hbm.at[pl.ds((first + cur) * _CB, _CB)],
                    osem).wait()
            for b in range(_CB):
                mb = plsc.load_gather(red_v, [mean_idx[b]])
                yb = plsc.load_gather(red_v, [y_idx[b]])
                for d in range(_NVEC):
                    v = plsc.load_gather(s_t, [st_idx[d][b]])
                    out_v[b, pl.ds(_NLANE * d, _NLANE)] = (v - mb) * yb
            pltpu.async_copy(
                out_v, o_hbm.at[pl.ds((first + cur) * _CB, _CB)], osem)

    for p in range(2):
        out_v, osem = obufs[p]
        pltpu.make_async_copy(
            out_v, o_hbm.at[pl.ds(first * _CB, _CB)], osem).wait()


_sc_embed_ln = pl.kernel(
    _sc_body,
    out_type=jax.ShapeDtypeStruct((_BATCH, _EMB_DIM), jnp.float32),
    mesh=plsc.VectorSubcoreMesh(core_axis_name="c", subcore_axis_name="s"),
    compiler_params=pltpu.CompilerParams(needs_layout_passes=False),
    scratch_types=[
        pltpu.VMEM((_IDX_PER_W,), jnp.int32),
        pltpu.VMEM((_IDX_PER_CHUNK, _EMB_DIM), jnp.float32),
        pltpu.VMEM((_IDX_PER_CHUNK, _EMB_DIM), jnp.float32),
        pltpu.VMEM((_CB, _EMB_DIM), jnp.float32),
        pltpu.VMEM((_CB, _EMB_DIM), jnp.float32),
        pltpu.VMEM((_EMB_DIM * _CB,), jnp.float32),
        pltpu.VMEM((4 * _NLANE,), jnp.float32),
        pltpu.SemaphoreType.DMA,
        pltpu.SemaphoreType.DMA,
        pltpu.SemaphoreType.DMA,
        pltpu.SemaphoreType.DMA,
    ],
)


@jax.jit
def kernel(x, tables):
    t_flat = tables.reshape(_N_FIELDS * _VOCAB, _EMB_DIM)
    idx = (x + _VOCAB * jnp.arange(_N_FIELDS, dtype=jnp.int32)[None, :]
           ).reshape(-1)
    return _sc_embed_ln(t_flat, idx)


# speedup vs baseline: 1.1942x; 1.1942x over previous
import jax
import jax.numpy as jnp
from jax import lax
from jax.experimental import pallas as pl
from jax.experimental.pallas import tpu as pltpu
from jax.experimental.pallas import tpu_sc as plsc

_EMB_DIM = 128
_N_FIELDS = 26
_VOCAB = 1000
_BATCH = 16384
_LN_EPS = 1e-5

_NC = 2
_NS = 16
_NW = _NC * _NS
_CB = 8
_ROWS_PER_W = _BATCH // _NW
_CHUNKS = _ROWS_PER_W // _CB
_IDX_PER_CHUNK = _N_FIELDS * _CB
_IDX_PER_W = _N_FIELDS * _ROWS_PER_W
_GATHER_SPLIT = 2
_IDX_PER_DMA = _IDX_PER_CHUNK // _GATHER_SPLIT
_NLANE = 16
_NVEC = _EMB_DIM // _NLANE


def _sc_body(t_hbm, idx_hbm, o_hbm, idx_all, rows0, rows1, out0, out1,
             s_t, red_v, sem0, sem1, osem0, osem1):
    wid = lax.axis_index("s") * _NC + lax.axis_index("c")
    first = wid * _CHUNKS
    bufs = ((rows0, sem0), (rows1, sem1))
    obufs = ((out0, osem0), (out1, osem1))

    pltpu.sync_copy(idx_hbm.at[pl.ds(wid * _IDX_PER_W, _IDX_PER_W)],
                    idx_all)

    def fire(local_chunk, p):
        rows_v, sem = bufs[p]
        for g in range(_GATHER_SPLIT):
            off = local_chunk * _IDX_PER_CHUNK + g * _IDX_PER_DMA
            pltpu.async_copy(
                t_hbm.at[idx_all.at[pl.ds(off, _IDX_PER_DMA)]],
                rows_v.at[pl.ds(g * _IDX_PER_DMA, _IDX_PER_DMA)],
                sem)

    def drain(local_chunk, p):
        rows_v, sem = bufs[p]
        for g in range(_GATHER_SPLIT):
            off = local_chunk * _IDX_PER_CHUNK + g * _IDX_PER_DMA
            pltpu.make_async_copy(
                t_hbm.at[idx_all.at[pl.ds(off, _IDX_PER_DMA)]],
                rows_v.at[pl.ds(g * _IDX_PER_DMA, _IDX_PER_DMA)],
                sem).wait()

    fire(0, 0)

    half = jnp.full((_NLANE,), 0.5, jnp.float32)
    three_half = jnp.full((_NLANE,), 1.5, jnp.float32)
    magic = jnp.full((_NLANE,), 0x5F3759DF, jnp.int32)
    one_i = jnp.full((_NLANE,), 1, jnp.int32)
    lanes = lax.iota(jnp.int32, _NLANE)
    lanes_cb = lanes * _CB
    st_idx = [[lanes_cb + (_NLANE * _CB * d + b) for b in range(_CB)]
              for d in range(_NVEC)]
    mean_idx = [jnp.full((_NLANE,), 2 * _NLANE + b, jnp.int32)
                for b in range(_CB)]
    y_idx = [jnp.full((_NLANE,), 3 * _NLANE + b, jnp.int32)
             for b in range(_CB)]
    xor_half = lanes ^ _CB
    xor_half_sq = xor_half + _NLANE

    @pl.loop(0, _CHUNKS, step=2)
    def chunk_loop(i):
        for p in range(2):
            rows_v = bufs[p][0]
            out_v, osem = obufs[p]
            cur = i + p
            nxt = cur + 1

            @pl.when(nxt < _CHUNKS)
            def _():
                fire(nxt, 1 - p)

            drain(cur, p)
            for b in range(_CB):
                base_row = b * _N_FIELDS

                def fbody(f, accs):
                    return tuple(
                        accs[d] + rows_v[base_row + f,
                                         pl.ds(_NLANE * d, _NLANE)]
                        for d in range(_NVEC))
                accs = lax.fori_loop(
                    0, _N_FIELDS, fbody,
                    tuple(jnp.zeros((_NLANE,), jnp.float32)
                          for _ in range(_NVEC)), unroll=1)
                for d in range(_NVEC):
                    plsc.store_scatter(s_t, [st_idx[d][b]], accs[d])
            def sbody(j, ms):
                v = s_t[pl.ds(j * _NLANE, _NLANE)]
                return ms[0] + v, ms[1] + v * v
            msum, msq = lax.fori_loop(
                0, _EMB_DIM * _CB // _NLANE, sbody,
                (jnp.zeros((_NLANE,), jnp.float32),
                 jnp.zeros((_NLANE,), jnp.float32)), unroll=4)
            red_v[pl.ds(0, _NLANE)] = msum
            red_v[pl.ds(_NLANE, _NLANE)] = msq
            msum = msum + plsc.load_gather(red_v, [xor_half])
            msq = msq + plsc.load_gather(red_v, [xor_half_sq])
            mean = msum * (1.0 / _EMB_DIM)
            var = msq * (1.0 / _EMB_DIM) - mean * mean + _LN_EPS
            y = plsc.bitcast(
                magic - lax.shift_right_logical(
                    plsc.bitcast(var, jnp.int32), one_i), jnp.float32)
            hx = half * var
            for _ in range(3):
                y = y * (three_half - hx * y * y)
            red_v[pl.ds(2 * _NLANE, _NLANE)] = mean
            red_v[pl.ds(3 * _NLANE, _NLANE)] = y
            @pl.when(cur >= 2)
            def _():
                pltpu.make_async_copy(
                    out_v, o_hbm.at[pl.ds((first + cur) * _CB, _CB)],
                    osem).wait()
            for b in range(_CB):
                mb = plsc.load_gather(red_v, [mean_idx[b]])
                yb = plsc.load_gather(red_v, [y_idx[b]])
                for d in range(_NVEC):
                    v = plsc.load_gather(s_t, [st_idx[d][b]])
                    out_v[b, pl.ds(_NLANE * d, _NLANE)] = (v - mb) * yb
            pltpu.async_copy(
                out_v, o_hbm.at[pl.ds((first + cur) * _CB, _CB)], osem)

    for p in range(2):
        out_v, osem = obufs[p]
        pltpu.make_async_copy(
            out_v, o_hbm.at[pl.ds(first * _CB, _CB)], osem).wait()


_sc_embed_ln = pl.kernel(
    _sc_body,
    out_type=jax.ShapeDtypeStruct((_BATCH, _EMB_DIM), jnp.float32),
    mesh=plsc.VectorSubcoreMesh(core_axis_name="c", subcore_axis_name="s"),
    compiler_params=pltpu.CompilerParams(needs_layout_passes=False),
    scratch_types=[
        pltpu.VMEM((_IDX_PER_W,), jnp.int32),
        pltpu.VMEM((_IDX_PER_CHUNK, _EMB_DIM), jnp.float32),
        pltpu.VMEM((_IDX_PER_CHUNK, _EMB_DIM), jnp.float32),
        pltpu.VMEM((_CB, _EMB_DIM), jnp.float32),
        pltpu.VMEM((_CB, _EMB_DIM), jnp.float32),
        pltpu.VMEM((_EMB_DIM * _CB,), jnp.float32),
        pltpu.VMEM((4 * _NLANE,), jnp.float32),
        pltpu.SemaphoreType.DMA,
        pltpu.SemaphoreType.DMA,
        pltpu.SemaphoreType.DMA,
        pltpu.SemaphoreType.DMA,
    ],
)


@jax.jit
def kernel(x, tables):
    t_flat = tables.reshape(_N_FIELDS * _VOCAB, _EMB_DIM)
    idx = (x + _VOCAB * jnp.arange(_N_FIELDS, dtype=jnp.int32)[None, :]
           ).reshape(-1)
    return _sc_embed_ln(t_flat, idx)


# speedup vs baseline: 1.2044x; 1.0086x over previous
import jax
import jax.numpy as jnp
from jax import lax
from jax.experimental import pallas as pl
from jax.experimental.pallas import tpu as pltpu
from jax.experimental.pallas import tpu_sc as plsc

_EMB_DIM = 128
_N_FIELDS = 26
_VOCAB = 1000
_BATCH = 16384
_LN_EPS = 1e-5

_NC = 2
_NS = 16
_NW = _NC * _NS
_CB = 8
_ROWS_PER_W = _BATCH // _NW
_CHUNKS = _ROWS_PER_W // _CB
_IDX_PER_CHUNK = _N_FIELDS * _CB
_IDX_PER_W = _N_FIELDS * _ROWS_PER_W
_GATHER_SPLIT = 2
_IDX_PER_DMA = _IDX_PER_CHUNK // _GATHER_SPLIT
_NLANE = 16
_NVEC = _EMB_DIM // _NLANE


def _sc_body(t_hbm, idx_hbm, o_hbm, idx_all, rows0, rows1, out0, out1,
             s_t, red_v, sem0, sem1, osem0, osem1):
    wid = lax.axis_index("s") * _NC + lax.axis_index("c")
    first = wid * _CHUNKS
    bufs = ((rows0, sem0), (rows1, sem1))
    obufs = ((out0, osem0), (out1, osem1))

    pltpu.sync_copy(idx_hbm.at[pl.ds(wid * _IDX_PER_W, _IDX_PER_W)],
                    idx_all)

    def fire(local_chunk, p):
        rows_v, sem = bufs[p]
        for g in range(_GATHER_SPLIT):
            off = local_chunk * _IDX_PER_CHUNK + g * _IDX_PER_DMA
            pltpu.async_copy(
                t_hbm.at[idx_all.at[pl.ds(off, _IDX_PER_DMA)]],
                rows_v.at[pl.ds(g * _IDX_PER_DMA, _IDX_PER_DMA)],
                sem)

    def drain(local_chunk, p):
        rows_v, sem = bufs[p]
        for g in range(_GATHER_SPLIT):
            off = local_chunk * _IDX_PER_CHUNK + g * _IDX_PER_DMA
            pltpu.make_async_copy(
                t_hbm.at[idx_all.at[pl.ds(off, _IDX_PER_DMA)]],
                rows_v.at[pl.ds(g * _IDX_PER_DMA, _IDX_PER_DMA)],
                sem).wait()

    fire(0, 0)

    half = jnp.full((_NLANE,), 0.5, jnp.float32)
    three_half = jnp.full((_NLANE,), 1.5, jnp.float32)
    magic = jnp.full((_NLANE,), 0x5F3759DF, jnp.int32)
    one_i = jnp.full((_NLANE,), 1, jnp.int32)
    lanes = lax.iota(jnp.int32, _NLANE)
    lanes_cb = lanes * _CB
    st_idx0 = [lanes_cb + _NLANE * _CB * d for d in range(_NVEC)]
    st_idx = [[st_idx0[d] + b for b in range(_CB)] for d in range(_NVEC)]
    mean_idx = [jnp.full((_NLANE,), 2 * _NLANE + b, jnp.int32)
                for b in range(_CB)]
    y_idx = [jnp.full((_NLANE,), 3 * _NLANE + b, jnp.int32)
             for b in range(_CB)]
    xor_half = lanes ^ _CB
    xor_half_sq = xor_half + _NLANE

    @pl.loop(0, _CHUNKS, step=2)
    def chunk_loop(i):
        for p in range(2):
            rows_v = bufs[p][0]
            out_v, osem = obufs[p]
            cur = i + p
            nxt = cur + 1

            @pl.when(nxt < _CHUNKS)
            def _():
                fire(nxt, 1 - p)

            drain(cur, p)
            def row_body(b, carry):
                base_row = b * _N_FIELDS

                def fbody(f, accs):
                    return tuple(
                        accs[d] + rows_v[base_row + f,
                                         pl.ds(_NLANE * d, _NLANE)]
                        for d in range(_NVEC))
                accs = lax.fori_loop(
                    0, _N_FIELDS, fbody,
                    tuple(jnp.zeros((_NLANE,), jnp.float32)
                          for _ in range(_NVEC)), unroll=1)
                bvec = jnp.full((_NLANE,), 1, jnp.int32) * b
                for d in range(_NVEC):
                    plsc.store_scatter(s_t, [st_idx0[d] + bvec], accs[d])
                return carry
            lax.fori_loop(0, _CB, row_body, 0)
            def sbody(j, ms):
                v = s_t[pl.ds(j * _NLANE, _NLANE)]
                return ms[0] + v, ms[1] + v * v
            msum, msq = lax.fori_loop(
                0, _EMB_DIM * _CB // _NLANE, sbody,
                (jnp.zeros((_NLANE,), jnp.float32),
                 jnp.zeros((_NLANE,), jnp.float32)), unroll=4)
            red_v[pl.ds(0, _NLANE)] = msum
            red_v[pl.ds(_NLANE, _NLANE)] = msq
            msum = msum + plsc.load_gather(red_v, [xor_half])
            msq = msq + plsc.load_gather(red_v, [xor_half_sq])
            mean = msum * (1.0 / _EMB_DIM)
            var = msq * (1.0 / _EMB_DIM) - mean * mean + _LN_EPS
            y = plsc.bitcast(
                magic - lax.shift_right_logical(
                    plsc.bitcast(var, jnp.int32), one_i), jnp.float32)
            hx = half * var
            for _ in range(3):
                y = y * (three_half - hx * y * y)
            red_v[pl.ds(2 * _NLANE, _NLANE)] = mean
            red_v[pl.ds(3 * _NLANE, _NLANE)] = y
            @pl.when(cur >= 2)
            def _():
                pltpu.make_async_copy(
                    out_v, o_hbm.at[pl.ds((first + cur) * _CB, _CB)],
                    osem).wait()
            for b in range(_CB):
                mb = plsc.load_gather(red_v, [mean_idx[b]])
                yb = plsc.load_gather(red_v, [y_idx[b]])
                for d in range(_NVEC):
                    v = plsc.load_gather(s_t, [st_idx[d][b]])
                    out_v[b, pl.ds(_NLANE * d, _NLANE)] = (v - mb) * yb
            pltpu.async_copy(
                out_v, o_hbm.at[pl.ds((first + cur) * _CB, _CB)], osem)

    for p in range(2):
        out_v, osem = obufs[p]
        pltpu.make_async_copy(
            out_v, o_hbm.at[pl.ds(first * _CB, _CB)], osem).wait()


_sc_embed_ln = pl.kernel(
    _sc_body,
    out_type=jax.ShapeDtypeStruct((_BATCH, _EMB_DIM), jnp.float32),
    mesh=plsc.VectorSubcoreMesh(core_axis_name="c", subcore_axis_name="s"),
    compiler_params=pltpu.CompilerParams(needs_layout_passes=False),
    scratch_types=[
        pltpu.VMEM((_IDX_PER_W,), jnp.int32),
        pltpu.VMEM((_IDX_PER_CHUNK, _EMB_DIM), jnp.float32),
        pltpu.VMEM((_IDX_PER_CHUNK, _EMB_DIM), jnp.float32),
        pltpu.VMEM((_CB, _EMB_DIM), jnp.float32),
        pltpu.VMEM((_CB, _EMB_DIM), jnp.float32),
        pltpu.VMEM((_EMB_DIM * _CB,), jnp.float32),
        pltpu.VMEM((4 * _NLANE,), jnp.float32),
        pltpu.SemaphoreType.DMA,
        pltpu.SemaphoreType.DMA,
        pltpu.SemaphoreType.DMA,
        pltpu.SemaphoreType.DMA,
    ],
)


@jax.jit
def kernel(x, tables):
    t_flat = tables.reshape(_N_FIELDS * _VOCAB, _EMB_DIM)
    idx = (x + _VOCAB * jnp.arange(_N_FIELDS, dtype=jnp.int32)[None, :]
           ).reshape(-1)
    return _sc_embed_ln(t_flat, idx)
